# baseline (speedup 1.0000x reference)
import functools

import jax
import jax.numpy as jnp
from jax import lax
from jax.experimental import pallas as pl
from jax.experimental.pallas import tpu as pltpu
from jax.experimental.pallas import tpu_sc as plsc

V_LIMIT = 25000
D = 128
B, L = 4096, 50
TOK = B * L
NW = 32
ROWS_PER_W = TOK // NW
TROWS = ROWS_PER_W // D
GB = 128
NB = ROWS_PER_W // GB
CROWS = NB + 2
TRASH = (CROWS - 1) * D
ZCH = 256
NZ = ROWS_PER_W // ZCH
OUT_PAD = 256

_mesh = plsc.VectorSubcoreMesh(core_axis_name="c", subcore_axis_name="s")


@functools.partial(
    pl.kernel,
    mesh=_mesh,
    compiler_params=pltpu.CompilerParams(needs_layout_passes=False),
    out_type=jax.ShapeDtypeStruct((TOK + OUT_PAD, D), jnp.float32),
    scratch_types=[
        pltpu.VMEM((TROWS, D), jnp.int32),
        pltpu.VMEM((CROWS, 1, D), jnp.int32),
        pltpu.VMEM((CROWS, 1, D), jnp.int32),
        pltpu.VMEM((2 * GB, D), jnp.float32),
        pltpu.VMEM((ZCH, D), jnp.float32),
        pltpu.SemaphoreType.DMA,
        pltpu.SemaphoreType.DMA,
        pltpu.SemaphoreType.DMA,
    ],
)
def _emb(tok_hbm, tbl_hbm, out_hbm, tok_v, cidx_v, cpos_v, rows_v, zero_v,
         zsem, gsem, wsem):
    wid = lax.axis_index("s") * 2 + lax.axis_index("c")
    row_base = wid * ROWS_PER_W
    lane = lax.iota(jnp.int32, 16)
    zero16 = lane * 0

    def zinit_body(r, carry):
        for c in range(D // 16):
            zero_v[r, pl.ds(c * 16, 16)] = jnp.zeros((16,), jnp.float32)
        return carry

    lax.fori_loop(0, ZCH, zinit_body, 0)

    def zfill(z, wait):
        cp = pltpu.make_async_copy(
            zero_v, out_hbm.at[pl.ds(row_base + z * ZCH, ZCH)], zsem
        )
        if wait:
            cp.wait()
        else:
            cp.start()
        return 0

    lax.fori_loop(0, NZ, lambda z, c: zfill(z, False), 0)

    pltpu.sync_copy(tok_hbm.at[wid], tok_v)

    def compact_body(i, cnt):
        for c in range(D // 16):
            t = tok_v[i, pl.ds(c * 16, 16)]
            valid = jnp.where(
                (t - 1).astype(jnp.uint32) < jnp.uint32(V_LIMIT - 1), 1, 0
            )
            ps = plsc.cumsum(valid)
            fdest = jnp.where(valid == 1, cnt + ps - 1, TRASH + lane)
            d0 = fdest >> 7
            d2 = fdest & (D - 1)
            pos = row_base + i * D + c * 16 + lane
            plsc.store_scatter(cidx_v, [d0, zero16, d2], t)
            plsc.store_scatter(cpos_v, [d0, zero16, d2], pos)
            cnt = cnt + ps[15]
        return cnt

    cnt = lax.fori_loop(0, TROWS, compact_body, jnp.int32(0))

    pad_idx = wid * 128 + lane * 8
    pad_pos = TOK + wid * 8 + (lane & 7)

    def pad_body(k, carry):
        f = cnt + k * 16 + lane
        d0 = f >> 7
        d2 = f & (D - 1)
        plsc.store_scatter(cidx_v, [d0, zero16, d2], pad_idx)
        plsc.store_scatter(cpos_v, [d0, zero16, d2], pad_pos)
        return carry

    lax.fori_loop(0, GB // 16, pad_body, 0)
    nb = jnp.maximum((cnt + GB - 1) // GB, 1)

    def gather(b, wait):
        cp = pltpu.make_async_copy(
            tbl_hbm.at[cidx_v.at[b, 0]],
            rows_v.at[pl.ds((b & 1) * GB, GB)],
            gsem,
        )
        if wait:
            cp.wait()
        else:
            cp.start()

    def scatter(b, wait):
        cp = pltpu.make_async_copy(
            rows_v.at[pl.ds((b & 1) * GB, GB)],
            out_hbm.at[cpos_v.at[b, 0]],
            wsem,
        )
        if wait:
            cp.wait()
        else:
            cp.start()

    gather(0, False)

    lax.fori_loop(0, NZ, lambda z, c: zfill(z, True), 0)

    def batch_body(b, carry):
        gather(b, True)

        @pl.when(b > 0)
        def _():
            scatter(b - 1, True)

        @pl.when(b + 1 < nb)
        def _():
            gather(b + 1, False)

        scatter(b, False)
        return carry

    lax.fori_loop(0, nb, batch_body, 0)
    scatter(nb - 1, True)


def kernel(tokens, table):
    tok3 = tokens.reshape(NW, TROWS, D)
    out = _emb(tok3, table)
    return out[:TOK].reshape(B, L, D)
